# baseline (speedup 1.0000x reference)
import functools

import jax
import jax.numpy as jnp
from jax import lax
from jax.experimental import pallas as pl
from jax.experimental.pallas import tpu as pltpu
from jax.experimental.pallas import tpu_sc as plsc

N = 10000
E = 320000
D = 128
G = 64
C = 10

NP = 10240
EP = 327680
ECHUNK = 128
EROWS = EP // ECHUNK
TILES = 32
ROWS_PER_TILE = EROWS // TILES
ACC_ROWS_PER_TILE = NP // 16
NBUF = 2
PHROWS = 40
CH0 = 80
CH1 = 80

def _seg_sum_sc(h, src_rows, dst_rows, zeros_hbm):
    mesh = plsc.VectorSubcoreMesh(core_axis_name="c", subcore_axis_name="s",
                                  num_cores=2, num_subcores=16)

    @functools.partial(
        pl.kernel,
        out_type=jax.ShapeDtypeStruct((2, NP, 128), jnp.float32),
        mesh=mesh,
        scratch_types=[
            pltpu.VMEM((PHROWS, ECHUNK), jnp.int32),
            pltpu.VMEM((PHROWS, ECHUNK), jnp.int32),
            pltpu.VMEM((NBUF, ECHUNK, 128), jnp.float32),
            pltpu.VMEM_SHARED((NP, 128), jnp.float32),
            [pltpu.SemaphoreType.DMA] * NBUF,
            [pltpu.SemaphoreType.DMA] * NBUF,
        ],
    )
    def k(h_hbm, src_hbm, dst_hbm, z_hbm, out_hbm, sidx, didx, bufs, acc,
          gsem, ssem):
        c = lax.axis_index("c")
        s = lax.axis_index("s")
        wid = c * 16 + s
        arow = s * ACC_ROWS_PER_TILE
        pltpu.sync_copy(z_hbm.at[pl.ds(arow, ACC_ROWS_PER_TILE)],
                        acc.at[pl.ds(arow, ACC_ROWS_PER_TILE)])
        plsc.subcore_barrier()

        nph = jnp.where(c == 0, CH0 // PHROWS, CH1 // PHROWS)
        tile_base = jnp.where(c == 0, s * CH0, 16 * CH0 + s * CH1)

        @pl.loop(0, nph)
        def _(ph):
            erow = tile_base + ph * PHROWS
            pltpu.sync_copy(src_hbm.at[pl.ds(erow, PHROWS)], sidx)
            pltpu.sync_copy(dst_hbm.at[pl.ds(erow, PHROWS)], didx)
            for b in range(NBUF):
                pltpu.async_copy(h_hbm.at[sidx.at[b]], bufs.at[b], gsem[b])

            @pl.loop(0, PHROWS, step=NBUF)
            def _(j):
                for b in range(NBUF):
                    pltpu.make_async_copy(h_hbm.at[sidx.at[j + b]],
                                          bufs.at[b], gsem[b]).wait()
                    pltpu.async_copy(bufs.at[b], acc.at[didx.at[j + b]],
                                     ssem[b], add=True)
                for b in range(NBUF):
                    pltpu.make_async_copy(bufs.at[b], acc.at[didx.at[j + b]],
                                          ssem[b]).wait()

                    @pl.when(j + NBUF + b < PHROWS)
                    def _(b=b):
                        pltpu.async_copy(h_hbm.at[sidx.at[j + NBUF + b]],
                                         bufs.at[b], gsem[b])

        plsc.subcore_barrier()
        pltpu.sync_copy(acc.at[pl.ds(arow, ACC_ROWS_PER_TILE)],
                        out_hbm.at[c, pl.ds(arow, ACC_ROWS_PER_TILE)])

    return k(h, src_rows, dst_rows, zeros_hbm)


_BR = 1024


def _layer_tc(p, h_in, w_rel_t, w_root_t, b):
    def body(p_ref, x_ref, wr_ref, wt_ref, b_ref, o_ref):
        agg = p_ref[0] + p_ref[1]
        h = (jnp.dot(agg, wr_ref[...], preferred_element_type=jnp.float32)
             + jnp.dot(x_ref[...], wt_ref[...], preferred_element_type=jnp.float32)
             + b_ref[...])
        o_ref[...] = jnp.maximum(h, 0.0)

    return pl.pallas_call(
        body,
        grid=(NP // _BR,),
        in_specs=[
            pl.BlockSpec((2, _BR, 128), lambda i: (0, i, 0)),
            pl.BlockSpec((_BR, 128), lambda i: (i, 0)),
            pl.BlockSpec((128, 128), lambda i: (0, 0)),
            pl.BlockSpec((128, 128), lambda i: (0, 0)),
            pl.BlockSpec((1, 128), lambda i: (0, 0)),
        ],
        out_specs=pl.BlockSpec((_BR, 128), lambda i: (i, 0)),
        out_shape=jax.ShapeDtypeStruct((NP, 128), jnp.float32),
    )(p, h_in, w_rel_t, w_root_t, b)


_FBR = 512
_FSTEPS = NP // _FBR


def _final_tc(p, h2, batch3, w3_t, wr3_t, b3, wl_t, bl):
    def body(p_ref, h_ref, bt_ref, w3_ref, wr_ref, b3_ref, wl_ref, bl_ref,
             o_ref, pool, cnt):
        i = pl.program_id(0)

        @pl.when(i == 0)
        def _():
            pool[...] = jnp.zeros_like(pool)
            cnt[...] = jnp.zeros_like(cnt)

        h3 = (jnp.dot(p_ref[0] + p_ref[1], w3_ref[...],
                      preferred_element_type=jnp.float32)
              + jnp.dot(h_ref[...], wr_ref[...],
                        preferred_element_type=jnp.float32)
              + b3_ref[...])
        bvals = bt_ref[0]
        gids = lax.broadcasted_iota(jnp.int32, (G, _FBR), 0)
        oh = jnp.where(gids == bvals, 1.0, 0.0)
        pool[...] += jnp.dot(oh, h3, preferred_element_type=jnp.float32)
        cnt[...] += jnp.sum(oh, axis=1, keepdims=True)

        @pl.when(i == _FSTEPS - 1)
        def _():
            denom = jnp.maximum(cnt[...], 1.0)
            pooled = pool[...] / denom
            o_ref[...] = (jnp.dot(pooled, wl_ref[...],
                                  preferred_element_type=jnp.float32)
                          + bl_ref[...])

    return pl.pallas_call(
        body,
        grid=(_FSTEPS,),
        in_specs=[
            pl.BlockSpec((2, _FBR, 128), lambda i: (0, i, 0)),
            pl.BlockSpec((_FBR, 128), lambda i: (i, 0)),
            pl.BlockSpec((1, 1, _FBR), lambda i: (i, 0, 0)),
            pl.BlockSpec((128, 128), lambda i: (0, 0)),
            pl.BlockSpec((128, 128), lambda i: (0, 0)),
            pl.BlockSpec((1, 128), lambda i: (0, 0)),
            pl.BlockSpec((128, C), lambda i: (0, 0)),
            pl.BlockSpec((1, C), lambda i: (0, 0)),
        ],
        out_specs=pl.BlockSpec((G, C), lambda i: (0, 0)),
        out_shape=jax.ShapeDtypeStruct((G, C), jnp.float32),
        scratch_shapes=[
            pltpu.VMEM((G, 128), jnp.float32),
            pltpu.VMEM((G, 1), jnp.float32),
        ],
    )(p, h2, batch3, w3_t, wr3_t, b3, wl_t, bl)


def kernel(x, edge_index, batch, W_rel1, b_rel1, W_root1, W_rel2, b_rel2,
           W_root2, W_rel3, b_rel3, W_root3, W_lin, b_lin):
    src = edge_index[0].astype(jnp.int32)
    dst = edge_index[1].astype(jnp.int32)
    pad = jnp.arange(EP - E, dtype=jnp.int32)
    src_rows = jnp.concatenate(
        [src, pad % N]).reshape(EROWS, ECHUNK)
    dst_rows = jnp.concatenate(
        [dst, N + pad % (NP - N)]).reshape(EROWS, ECHUNK)
    xp = jnp.concatenate([x, jnp.zeros((NP - N, 128), jnp.float32)])
    batch3 = jnp.concatenate(
        [batch.astype(jnp.int32), jnp.full((NP - N,), G, jnp.int32)]
    ).reshape(_FSTEPS, 1, _FBR)
    zeros_hbm = jnp.zeros((NP, 128), jnp.float32)

    p1 = _seg_sum_sc(xp, src_rows, dst_rows, zeros_hbm)
    h1 = _layer_tc(p1, xp, W_rel1.T, W_root1.T, b_rel1.reshape(1, 128))
    p2 = _seg_sum_sc(h1, src_rows, dst_rows, zeros_hbm)
    h2 = _layer_tc(p2, h1, W_rel2.T, W_root2.T, b_rel2.reshape(1, 128))
    p3 = _seg_sum_sc(h2, src_rows, dst_rows, zeros_hbm)
    out = _final_tc(p3, h2, batch3, W_rel3.T, W_root3.T,
                    b_rel3.reshape(1, 128), W_lin.T, b_lin.reshape(1, C))
    return out


# speedup vs baseline: 1.1660x; 1.1660x over previous
import functools

import jax
import jax.numpy as jnp
from jax import lax
from jax.experimental import pallas as pl
from jax.experimental.pallas import tpu as pltpu
from jax.experimental.pallas import tpu_sc as plsc

N = 10000
E = 320000
D = 128
G = 64
C = 10

NP = 10240
EP = 327680
ECHUNK = 80
EROWS = EP // ECHUNK
TILES = 32
ROWS_PER_TILE = EROWS // TILES
ACC_ROWS_PER_TILE = NP // 16
NBUF = 4
PHROWS = 32
CH0 = 128
CH1 = 128

def _seg_sum_sc(h, src_rows, dst_rows, zeros_hbm):
    mesh = plsc.VectorSubcoreMesh(core_axis_name="c", subcore_axis_name="s",
                                  num_cores=2, num_subcores=16)

    @functools.partial(
        pl.kernel,
        out_type=jax.ShapeDtypeStruct((2, NP, 128), jnp.float32),
        mesh=mesh,
        scratch_types=[
            pltpu.VMEM((PHROWS, ECHUNK), jnp.int32),
            pltpu.VMEM((PHROWS, ECHUNK), jnp.int32),
            pltpu.VMEM((NBUF, ECHUNK, 128), jnp.float32),
            pltpu.VMEM_SHARED((NP, 128), jnp.float32),
            [pltpu.SemaphoreType.DMA] * NBUF,
            [pltpu.SemaphoreType.DMA] * NBUF,
        ],
    )
    def k(h_hbm, src_hbm, dst_hbm, z_hbm, out_hbm, sidx, didx, bufs, acc,
          gsem, ssem):
        c = lax.axis_index("c")
        s = lax.axis_index("s")
        wid = c * 16 + s
        arow = s * ACC_ROWS_PER_TILE
        pltpu.sync_copy(z_hbm.at[pl.ds(arow, ACC_ROWS_PER_TILE)],
                        acc.at[pl.ds(arow, ACC_ROWS_PER_TILE)])
        plsc.subcore_barrier()

        nph = jnp.where(c == 0, CH0 // PHROWS, CH1 // PHROWS)
        tile_base = jnp.where(c == 0, s * CH0, 16 * CH0 + s * CH1)

        @pl.loop(0, nph)
        def _(ph):
            erow = tile_base + ph * PHROWS
            pltpu.sync_copy(src_hbm.at[pl.ds(erow, PHROWS)], sidx)
            pltpu.sync_copy(dst_hbm.at[pl.ds(erow, PHROWS)], didx)
            for b in range(NBUF):
                pltpu.async_copy(h_hbm.at[sidx.at[b]], bufs.at[b], gsem[b])

            @pl.loop(0, PHROWS, step=NBUF)
            def _(j):
                for b in range(NBUF):
                    pltpu.make_async_copy(h_hbm.at[sidx.at[j + b]],
                                          bufs.at[b], gsem[b]).wait()
                    pltpu.async_copy(bufs.at[b], acc.at[didx.at[j + b]],
                                     ssem[b], add=True)
                for b in range(NBUF):
                    pltpu.make_async_copy(bufs.at[b], acc.at[didx.at[j + b]],
                                          ssem[b]).wait()

                    @pl.when(j + NBUF + b < PHROWS)
                    def _(b=b):
                        pltpu.async_copy(h_hbm.at[sidx.at[j + NBUF + b]],
                                         bufs.at[b], gsem[b])

        plsc.subcore_barrier()
        pltpu.sync_copy(acc.at[pl.ds(arow, ACC_ROWS_PER_TILE)],
                        out_hbm.at[c, pl.ds(arow, ACC_ROWS_PER_TILE)])

    return k(h, src_rows, dst_rows, zeros_hbm)


_BR = 1024


def _layer_tc(p, h_in, w_rel_t, w_root_t, b):
    def body(p_ref, x_ref, wr_ref, wt_ref, b_ref, o_ref):
        agg = p_ref[0] + p_ref[1]
        h = (jnp.dot(agg, wr_ref[...], preferred_element_type=jnp.float32)
             + jnp.dot(x_ref[...], wt_ref[...], preferred_element_type=jnp.float32)
             + b_ref[...])
        o_ref[...] = jnp.maximum(h, 0.0)

    return pl.pallas_call(
        body,
        grid=(NP // _BR,),
        in_specs=[
            pl.BlockSpec((2, _BR, 128), lambda i: (0, i, 0)),
            pl.BlockSpec((_BR, 128), lambda i: (i, 0)),
            pl.BlockSpec((128, 128), lambda i: (0, 0)),
            pl.BlockSpec((128, 128), lambda i: (0, 0)),
            pl.BlockSpec((1, 128), lambda i: (0, 0)),
        ],
        out_specs=pl.BlockSpec((_BR, 128), lambda i: (i, 0)),
        out_shape=jax.ShapeDtypeStruct((NP, 128), jnp.float32),
    )(p, h_in, w_rel_t, w_root_t, b)


_FBR = 512
_FSTEPS = NP // _FBR


def _final_tc(p, h2, batch3, w3_t, wr3_t, b3, wl_t, bl):
    def body(p_ref, h_ref, bt_ref, w3_ref, wr_ref, b3_ref, wl_ref, bl_ref,
             o_ref, pool, cnt):
        i = pl.program_id(0)

        @pl.when(i == 0)
        def _():
            pool[...] = jnp.zeros_like(pool)
            cnt[...] = jnp.zeros_like(cnt)

        h3 = (jnp.dot(p_ref[0] + p_ref[1], w3_ref[...],
                      preferred_element_type=jnp.float32)
              + jnp.dot(h_ref[...], wr_ref[...],
                        preferred_element_type=jnp.float32)
              + b3_ref[...])
        bvals = bt_ref[0]
        gids = lax.broadcasted_iota(jnp.int32, (G, _FBR), 0)
        oh = jnp.where(gids == bvals, 1.0, 0.0)
        pool[...] += jnp.dot(oh, h3, preferred_element_type=jnp.float32)
        cnt[...] += jnp.sum(oh, axis=1, keepdims=True)

        @pl.when(i == _FSTEPS - 1)
        def _():
            denom = jnp.maximum(cnt[...], 1.0)
            pooled = pool[...] / denom
            o_ref[...] = (jnp.dot(pooled, wl_ref[...],
                                  preferred_element_type=jnp.float32)
                          + bl_ref[...])

    return pl.pallas_call(
        body,
        grid=(_FSTEPS,),
        in_specs=[
            pl.BlockSpec((2, _FBR, 128), lambda i: (0, i, 0)),
            pl.BlockSpec((_FBR, 128), lambda i: (i, 0)),
            pl.BlockSpec((1, 1, _FBR), lambda i: (i, 0, 0)),
            pl.BlockSpec((128, 128), lambda i: (0, 0)),
            pl.BlockSpec((128, 128), lambda i: (0, 0)),
            pl.BlockSpec((1, 128), lambda i: (0, 0)),
            pl.BlockSpec((128, C), lambda i: (0, 0)),
            pl.BlockSpec((1, C), lambda i: (0, 0)),
        ],
        out_specs=pl.BlockSpec((G, C), lambda i: (0, 0)),
        out_shape=jax.ShapeDtypeStruct((G, C), jnp.float32),
        scratch_shapes=[
            pltpu.VMEM((G, 128), jnp.float32),
            pltpu.VMEM((G, 1), jnp.float32),
        ],
    )(p, h2, batch3, w3_t, wr3_t, b3, wl_t, bl)


def kernel(x, edge_index, batch, W_rel1, b_rel1, W_root1, W_rel2, b_rel2,
           W_root2, W_rel3, b_rel3, W_root3, W_lin, b_lin):
    src = edge_index[0].astype(jnp.int32)
    dst = edge_index[1].astype(jnp.int32)
    pad = jnp.arange(EP - E, dtype=jnp.int32)
    src_rows = jnp.concatenate(
        [src, pad % N]).reshape(EROWS, ECHUNK)
    dst_rows = jnp.concatenate(
        [dst, N + pad % (NP - N)]).reshape(EROWS, ECHUNK)
    xp = jnp.concatenate([x, jnp.zeros((NP - N, 128), jnp.float32)])
    batch3 = jnp.concatenate(
        [batch.astype(jnp.int32), jnp.full((NP - N,), G, jnp.int32)]
    ).reshape(_FSTEPS, 1, _FBR)
    zeros_hbm = jnp.zeros((NP, 128), jnp.float32)

    p1 = _seg_sum_sc(xp, src_rows, dst_rows, zeros_hbm)
    h1 = _layer_tc(p1, xp, W_rel1.T, W_root1.T, b_rel1.reshape(1, 128))
    p2 = _seg_sum_sc(h1, src_rows, dst_rows, zeros_hbm)
    h2 = _layer_tc(p2, h1, W_rel2.T, W_root2.T, b_rel2.reshape(1, 128))
    p3 = _seg_sum_sc(h2, src_rows, dst_rows, zeros_hbm)
    out = _final_tc(p3, h2, batch3, W_rel3.T, W_root3.T,
                    b_rel3.reshape(1, 128), W_lin.T, b_lin.reshape(1, C))
    return out
